# Initial kernel scaffold; baseline (speedup 1.0000x reference)
import functools

import jax
import jax.numpy as jnp
from jax import lax
from jax.experimental import pallas as pl
from jax.experimental.pallas import tpu as pltpu
from jax.experimental.pallas import tpu_sc as plsc

N = 10000
NP = 10240
D = 128
H = 128
C = 40
E = 320000

NC = 2
NS = 16
NW = NC * NS
K = 80
ET = E // NW
NCH = ET // K
RPT = NP // NS
DW = 8

_mesh = plsc.VectorSubcoreMesh(core_axis_name="c", subcore_axis_name="s")


def _deg_body(dst2_hbm, ones_hbm, zeros_hbm, out_hbm, ones_v, dbuf, acc_sh):
    c = lax.axis_index("c")
    s = lax.axis_index("s")
    w = c * NS + s
    pltpu.sync_copy(ones_hbm, ones_v)
    pltpu.sync_copy(zeros_hbm.at[pl.ds(s * RPT, RPT)],
                    acc_sh.at[pl.ds(s * RPT, RPT)])
    pltpu.sync_copy(dst2_hbm.at[pl.ds(w * NCH, NCH)], dbuf)
    plsc.subcore_barrier()

    def chunk(j, carry):
        pltpu.sync_copy(ones_v, acc_sh.at[dbuf.at[j]], add=True)
        return carry

    lax.fori_loop(0, NCH, chunk, 0)
    plsc.subcore_barrier()
    pltpu.sync_copy(acc_sh.at[pl.ds(s * RPT, RPT)],
                    out_hbm.at[pl.ds(c * NP + s * RPT, RPT)])


_deg = pl.kernel(
    _deg_body,
    out_type=jax.ShapeDtypeStruct((2 * NP, DW), jnp.float32),
    mesh=_mesh,
    scratch_types=[
        pltpu.VMEM((K, DW), jnp.float32),
        pltpu.VMEM((NCH, K), jnp.int32),
        pltpu.VMEM_SHARED((NP, DW), jnp.float32),
    ],
)


def _agg_body(hs_hbm, src2_hbm, dst2_hbm, zeros_hbm, out_hbm,
              sbuf, dbuf, rows_v, acc_sh, sem):
    c = lax.axis_index("c")
    s = lax.axis_index("s")
    w = c * NS + s
    pltpu.sync_copy(zeros_hbm.at[pl.ds(s * RPT, RPT)],
                    acc_sh.at[pl.ds(s * RPT, RPT)])
    pltpu.sync_copy(src2_hbm.at[pl.ds(w * NCH, NCH)], sbuf)
    pltpu.sync_copy(dst2_hbm.at[pl.ds(w * NCH, NCH)], dbuf)
    plsc.subcore_barrier()

    def chunk(j, carry):
        pltpu.async_copy(hs_hbm.at[sbuf.at[j]], rows_v, sem).wait()
        pltpu.sync_copy(rows_v, acc_sh.at[dbuf.at[j]], add=True)
        return carry

    lax.fori_loop(0, NCH, chunk, 0)
    plsc.subcore_barrier()
    pltpu.sync_copy(acc_sh.at[pl.ds(s * RPT, RPT)],
                    out_hbm.at[pl.ds(c * NP + s * RPT, RPT)])


_agg = pl.kernel(
    _agg_body,
    out_type=jax.ShapeDtypeStruct((2 * NP, H), jnp.float32),
    mesh=_mesh,
    scratch_types=[
        pltpu.VMEM((NCH, K), jnp.int32),
        pltpu.VMEM((NCH, K), jnp.int32),
        pltpu.VMEM((K, H), jnp.float32),
        pltpu.VMEM_SHARED((NP, H), jnp.float32),
        pltpu.SemaphoreType.DMA,
    ],
)


R = 1024
GRID = NP // R


def _mm_scale_kernel(x_ref, w_ref, d0_ref, d1_ref, hs_ref, dinv_ref):
    deg = d0_ref[...] + d1_ref[...] + 1.0
    dinv = lax.rsqrt(deg)
    h = jnp.dot(x_ref[...], w_ref[...], preferred_element_type=jnp.float32)
    hs_ref[...] = h * dinv
    dinv_ref[...] = jnp.broadcast_to(dinv, (R, H))


def _mm_scale(xp, W1, d0, d1):
    return pl.pallas_call(
        _mm_scale_kernel,
        grid=(GRID,),
        in_specs=[
            pl.BlockSpec((R, D), lambda i: (i, 0)),
            pl.BlockSpec((D, H), lambda i: (0, 0)),
            pl.BlockSpec((R, 1), lambda i: (i, 0)),
            pl.BlockSpec((R, 1), lambda i: (i, 0)),
        ],
        out_specs=[
            pl.BlockSpec((R, H), lambda i: (i, 0)),
            pl.BlockSpec((R, H), lambda i: (i, 0)),
        ],
        out_shape=[
            jax.ShapeDtypeStruct((NP, H), jnp.float32),
            jax.ShapeDtypeStruct((NP, H), jnp.float32),
        ],
    )(xp, W1, d0, d1)


def _mid_kernel(p0_ref, p1_ref, hs_ref, dinv_ref, b1_ref, g2_ref):
    agg = p0_ref[...] + p1_ref[...] + hs_ref[...]
    pre = agg * dinv_ref[...] + b1_ref[...]
    g2_ref[...] = jnp.maximum(pre, 0.0) * dinv_ref[...]


def _mid(p0, p1, hs, dinvB, b1):
    return pl.pallas_call(
        _mid_kernel,
        grid=(GRID,),
        in_specs=[
            pl.BlockSpec((R, H), lambda i: (i, 0)),
            pl.BlockSpec((R, H), lambda i: (i, 0)),
            pl.BlockSpec((R, H), lambda i: (i, 0)),
            pl.BlockSpec((R, H), lambda i: (i, 0)),
            pl.BlockSpec((1, H), lambda i: (0, 0)),
        ],
        out_specs=pl.BlockSpec((R, H), lambda i: (i, 0)),
        out_shape=jax.ShapeDtypeStruct((NP, H), jnp.float32),
    )(p0, p1, hs, dinvB, b1)


def _final_kernel(q0_ref, q1_ref, g2_ref, dinv_ref, w2_ref, b2_ref, out_ref):
    t = (q0_ref[...] + q1_ref[...] + g2_ref[...]) * dinv_ref[...]
    logits = jnp.dot(t, w2_ref[...], preferred_element_type=jnp.float32)
    logits = logits + b2_ref[...]
    m = jnp.max(logits, axis=1, keepdims=True)
    y = logits - m
    lse = jnp.log(jnp.sum(jnp.exp(y), axis=1, keepdims=True))
    out_ref[...] = y - lse


def _final(q0, q1, g2, dinvB, W2, b2):
    return pl.pallas_call(
        _final_kernel,
        grid=(GRID,),
        in_specs=[
            pl.BlockSpec((R, H), lambda i: (i, 0)),
            pl.BlockSpec((R, H), lambda i: (i, 0)),
            pl.BlockSpec((R, H), lambda i: (i, 0)),
            pl.BlockSpec((R, H), lambda i: (i, 0)),
            pl.BlockSpec((H, C), lambda i: (0, 0)),
            pl.BlockSpec((1, C), lambda i: (0, 0)),
        ],
        out_specs=pl.BlockSpec((R, C), lambda i: (i, 0)),
        out_shape=jax.ShapeDtypeStruct((NP, C), jnp.float32),
    )(q0, q1, g2, dinvB, W2, b2)


@jax.jit
def kernel(x, edge_index, W1, b1, W2, b2):
    ei = edge_index.astype(jnp.int32)
    src2 = ei[0].reshape(E // K, K)
    dst2 = ei[1].reshape(E // K, K)
    xp = jnp.pad(x, ((0, NP - N), (0, 0)))

    ones8 = jnp.ones((K, DW), jnp.float32)
    zeros8 = jnp.zeros((NP, DW), jnp.float32)
    zerosD = jnp.zeros((NP, H), jnp.float32)

    degp = _deg(dst2, ones8, zeros8)
    d0 = degp[:NP, 0:1]
    d1 = degp[NP:, 0:1]

    hs, dinvB = _mm_scale(xp, W1, d0, d1)

    p = _agg(hs, src2, dst2, zerosD)
    g2 = _mid(p[:NP], p[NP:], hs, dinvB, b1.reshape(1, H))

    q = _agg(g2, src2, dst2, zerosD)
    out = _final(q[:NP], q[NP:], g2, dinvB, W2, b2.reshape(1, C))
    return out[:N]


# speedup vs baseline: 16.7345x; 16.7345x over previous
import functools

import jax
import jax.numpy as jnp
from jax import lax
from jax.experimental import pallas as pl
from jax.experimental.pallas import tpu as pltpu
from jax.experimental.pallas import tpu_sc as plsc

N = 10000
NP = 10240
D = 128
H = 128
C = 40
E = 320000

NC = 2
NS = 16
NW = NC * NS
K = 80
ET = E // NW
NCH = ET // K
RPT = NP // NS
DW = 16

_mesh = plsc.VectorSubcoreMesh(core_axis_name="c", subcore_axis_name="s")


def _deg_body(dst2_hbm, ones_hbm, zeros_hbm, out_hbm, ones_v, dbuf, acc_sh):
    c = lax.axis_index("c")
    s = lax.axis_index("s")
    w = c * NS + s
    pltpu.sync_copy(ones_hbm, ones_v)
    pltpu.sync_copy(zeros_hbm.at[pl.ds(s * RPT, RPT)],
                    acc_sh.at[pl.ds(s * RPT, RPT)])
    pltpu.sync_copy(dst2_hbm.at[w], dbuf)
    plsc.subcore_barrier()

    def chunk(j, carry):
        pltpu.sync_copy(ones_v, acc_sh.at[dbuf.at[j]], add=True)
        return carry

    lax.fori_loop(0, NCH, chunk, 0)
    plsc.subcore_barrier()
    pltpu.sync_copy(acc_sh.at[pl.ds(s * RPT, RPT)],
                    out_hbm.at[pl.ds(c * NP + s * RPT, RPT)])


_deg = pl.kernel(
    _deg_body,
    out_type=jax.ShapeDtypeStruct((2 * NP, H), jnp.float32),
    mesh=_mesh,
    scratch_types=[
        pltpu.VMEM((K, H), jnp.float32),
        pltpu.VMEM((NCH, K), jnp.int32),
        pltpu.VMEM_SHARED((NP, H), jnp.float32),
    ],
)


def _agg_body(hs_hbm, src2_hbm, dst2_hbm, zeros_hbm, out_hbm,
              sbuf, dbuf, rows_v, acc_sh, sem):
    c = lax.axis_index("c")
    s = lax.axis_index("s")
    w = c * NS + s
    pltpu.sync_copy(zeros_hbm.at[pl.ds(s * RPT, RPT)],
                    acc_sh.at[pl.ds(s * RPT, RPT)])
    pltpu.sync_copy(src2_hbm.at[w], sbuf)
    pltpu.sync_copy(dst2_hbm.at[w], dbuf)
    plsc.subcore_barrier()

    def chunk(j, carry):
        pltpu.async_copy(hs_hbm.at[sbuf.at[j]], rows_v, sem).wait()
        pltpu.sync_copy(rows_v, acc_sh.at[dbuf.at[j]], add=True)
        return carry

    lax.fori_loop(0, NCH, chunk, 0)
    plsc.subcore_barrier()
    pltpu.sync_copy(acc_sh.at[pl.ds(s * RPT, RPT)],
                    out_hbm.at[pl.ds(c * NP + s * RPT, RPT)])


_agg = pl.kernel(
    _agg_body,
    out_type=jax.ShapeDtypeStruct((2 * NP, H), jnp.float32),
    mesh=_mesh,
    scratch_types=[
        pltpu.VMEM((NCH, K), jnp.int32),
        pltpu.VMEM((NCH, K), jnp.int32),
        pltpu.VMEM((K, H), jnp.float32),
        pltpu.VMEM_SHARED((NP, H), jnp.float32),
        pltpu.SemaphoreType.DMA,
    ],
)


R = 1024
GRID = NP // R


def _mm_scale_kernel(x_ref, w_ref, d0_ref, d1_ref, hs_ref, dinv_ref):
    deg = d0_ref[...] + d1_ref[...] + 1.0
    dinv = lax.rsqrt(deg)
    h = jnp.dot(x_ref[...], w_ref[...], preferred_element_type=jnp.float32)
    hs_ref[...] = h * dinv
    dinv_ref[...] = jnp.broadcast_to(dinv, (R, H))


def _mm_scale(xp, W1, d0, d1):
    return pl.pallas_call(
        _mm_scale_kernel,
        grid=(GRID,),
        in_specs=[
            pl.BlockSpec((R, D), lambda i: (i, 0)),
            pl.BlockSpec((D, H), lambda i: (0, 0)),
            pl.BlockSpec((R, 1), lambda i: (i, 0)),
            pl.BlockSpec((R, 1), lambda i: (i, 0)),
        ],
        out_specs=[
            pl.BlockSpec((R, H), lambda i: (i, 0)),
            pl.BlockSpec((R, H), lambda i: (i, 0)),
        ],
        out_shape=[
            jax.ShapeDtypeStruct((NP, H), jnp.float32),
            jax.ShapeDtypeStruct((NP, H), jnp.float32),
        ],
    )(xp, W1, d0, d1)


def _mid_kernel(p0_ref, p1_ref, hs_ref, dinv_ref, b1_ref, g2_ref):
    agg = p0_ref[...] + p1_ref[...] + hs_ref[...]
    pre = agg * dinv_ref[...] + b1_ref[...]
    g2_ref[...] = jnp.maximum(pre, 0.0) * dinv_ref[...]


def _mid(p0, p1, hs, dinvB, b1):
    return pl.pallas_call(
        _mid_kernel,
        grid=(GRID,),
        in_specs=[
            pl.BlockSpec((R, H), lambda i: (i, 0)),
            pl.BlockSpec((R, H), lambda i: (i, 0)),
            pl.BlockSpec((R, H), lambda i: (i, 0)),
            pl.BlockSpec((R, H), lambda i: (i, 0)),
            pl.BlockSpec((1, H), lambda i: (0, 0)),
        ],
        out_specs=pl.BlockSpec((R, H), lambda i: (i, 0)),
        out_shape=jax.ShapeDtypeStruct((NP, H), jnp.float32),
    )(p0, p1, hs, dinvB, b1)


def _final_kernel(q0_ref, q1_ref, g2_ref, dinv_ref, w2_ref, b2_ref, out_ref):
    t = (q0_ref[...] + q1_ref[...] + g2_ref[...]) * dinv_ref[...]
    logits = jnp.dot(t, w2_ref[...], preferred_element_type=jnp.float32)
    logits = logits + b2_ref[...]
    m = jnp.max(logits, axis=1, keepdims=True)
    y = logits - m
    lse = jnp.log(jnp.sum(jnp.exp(y), axis=1, keepdims=True))
    out_ref[...] = y - lse


def _final(q0, q1, g2, dinvB, W2, b2):
    return pl.pallas_call(
        _final_kernel,
        grid=(GRID,),
        in_specs=[
            pl.BlockSpec((R, H), lambda i: (i, 0)),
            pl.BlockSpec((R, H), lambda i: (i, 0)),
            pl.BlockSpec((R, H), lambda i: (i, 0)),
            pl.BlockSpec((R, H), lambda i: (i, 0)),
            pl.BlockSpec((H, C), lambda i: (0, 0)),
            pl.BlockSpec((1, C), lambda i: (0, 0)),
        ],
        out_specs=pl.BlockSpec((R, C), lambda i: (i, 0)),
        out_shape=jax.ShapeDtypeStruct((NP, C), jnp.float32),
    )(q0, q1, g2, dinvB, W2, b2)


@jax.jit
def kernel(x, edge_index, W1, b1, W2, b2):
    ei = edge_index.astype(jnp.int32)
    src2 = ei[0].reshape(NW, NCH, K)
    dst2 = ei[1].reshape(NW, NCH, K)
    xp = jnp.pad(x, ((0, NP - N), (0, 0)))

    zerosD = jnp.zeros((NP, H), jnp.float32)

    onesK = jnp.ones((K, H), jnp.float32)
    degp = _deg(dst2, onesK, zerosD)
    d0 = degp[:NP, 0:1]
    d1 = degp[NP:, 0:1]

    hs, dinvB = _mm_scale(xp, W1, d0, d1)

    p = _agg(hs, src2, dst2, zerosD)
    g2 = _mid(p[:NP], p[NP:], hs, dinvB, b1.reshape(1, H))

    q = _agg(g2, src2, dst2, zerosD)
    out = _final(q[:NP], q[NP:], g2, dinvB, W2, b2.reshape(1, C))
    return out[:N]


# speedup vs baseline: 24.0316x; 1.4361x over previous
import functools

import jax
import jax.numpy as jnp
from jax import lax
from jax.experimental import pallas as pl
from jax.experimental.pallas import tpu as pltpu
from jax.experimental.pallas import tpu_sc as plsc

N = 10000
NP = 10240
D = 128
H = 128
C = 40
E = 320000

NC = 2
NS = 16
NW = NC * NS
K = 80
ET = E // NW
NCH = ET // K
RPT = NP // NS
DW = 16

_mesh = plsc.VectorSubcoreMesh(core_axis_name="c", subcore_axis_name="s")


def _deg_body(dst2_hbm, ones_hbm, zeros_hbm, out_hbm, ones_v, dbuf, acc_sh):
    c = lax.axis_index("c")
    s = lax.axis_index("s")
    w = c * NS + s
    pltpu.sync_copy(ones_hbm, ones_v)
    pltpu.sync_copy(zeros_hbm.at[pl.ds(s * RPT, RPT)],
                    acc_sh.at[pl.ds(s * RPT, RPT)])
    pltpu.sync_copy(dst2_hbm.at[w], dbuf)
    plsc.subcore_barrier()

    def chunk(j, carry):
        pltpu.sync_copy(ones_v, acc_sh.at[dbuf.at[j]], add=True)
        return carry

    lax.fori_loop(0, NCH, chunk, 0)
    plsc.subcore_barrier()
    pltpu.sync_copy(acc_sh.at[pl.ds(s * RPT, RPT)],
                    out_hbm.at[pl.ds(c * NP + s * RPT, RPT)])


_deg = pl.kernel(
    _deg_body,
    out_type=jax.ShapeDtypeStruct((2 * NP, H), jnp.float32),
    mesh=_mesh,
    scratch_types=[
        pltpu.VMEM((K, H), jnp.float32),
        pltpu.VMEM((NCH, K), jnp.int32),
        pltpu.VMEM_SHARED((NP, H), jnp.float32),
    ],
)


_RB = K * H * 4
_IB = K * 4


def _agg_body(hs_hbm, src1_hbm, dst1_hbm, zeros_hbm, out_hbm,
              sbuf, d0, d1, rows0, rows1, acc_sh,
              semg0, semg1, semi0, semi1):
    c = lax.axis_index("c")
    s = lax.axis_index("s")
    w = c * NS + s
    eb = w * ET
    pltpu.sync_copy(zeros_hbm.at[pl.ds(s * RPT, RPT)],
                    acc_sh.at[pl.ds(s * RPT, RPT)])
    pltpu.sync_copy(src1_hbm.at[pl.ds(eb, ET)], sbuf)
    pltpu.async_copy(dst1_hbm.at[pl.ds(eb, K)], d0, semi0)
    pltpu.async_copy(dst1_hbm.at[pl.ds(eb + K, K)], d1, semi1)
    pltpu.async_copy(hs_hbm.at[sbuf.at[pl.ds(0, K)]], rows0, semg0)
    pltpu.async_copy(hs_hbm.at[sbuf.at[pl.ds(K, K)]], rows1, semg1)
    plsc.subcore_barrier()

    def pair(p, carry):
        j0 = 2 * p
        j1 = j0 + 1
        pltpu.make_async_copy(hs_hbm.at[pl.ds(0, K)], rows0, semg0).wait()
        pltpu.make_async_copy(dst1_hbm.at[pl.ds(0, K)], d0, semi0).wait()
        pltpu.sync_copy(rows0, acc_sh.at[d0], add=True)

        @pl.when(j0 + 2 < NCH)
        def _():
            pltpu.async_copy(dst1_hbm.at[pl.ds(eb + (j0 + 2) * K, K)],
                             d0, semi0)
            pltpu.async_copy(hs_hbm.at[sbuf.at[pl.ds((j0 + 2) * K, K)]],
                             rows0, semg0)

        pltpu.make_async_copy(hs_hbm.at[pl.ds(0, K)], rows1, semg1).wait()
        pltpu.make_async_copy(dst1_hbm.at[pl.ds(0, K)], d1, semi1).wait()
        pltpu.sync_copy(rows1, acc_sh.at[d1], add=True)

        @pl.when(j1 + 2 < NCH)
        def _():
            pltpu.async_copy(dst1_hbm.at[pl.ds(eb + (j1 + 2) * K, K)],
                             d1, semi1)
            pltpu.async_copy(hs_hbm.at[sbuf.at[pl.ds((j1 + 2) * K, K)]],
                             rows1, semg1)

        return carry

    lax.fori_loop(0, NCH // 2, pair, 0)
    if NCH % 2:
        pltpu.make_async_copy(hs_hbm.at[pl.ds(0, K)], rows0, semg0).wait()
        pltpu.make_async_copy(dst1_hbm.at[pl.ds(0, K)], d0, semi0).wait()
        pltpu.sync_copy(rows0, acc_sh.at[d0], add=True)
    plsc.subcore_barrier()
    pltpu.sync_copy(acc_sh.at[pl.ds(s * RPT, RPT)],
                    out_hbm.at[pl.ds(c * NP + s * RPT, RPT)])


_agg = pl.kernel(
    _agg_body,
    out_type=jax.ShapeDtypeStruct((2 * NP, H), jnp.float32),
    mesh=_mesh,
    scratch_types=[
        pltpu.VMEM((ET,), jnp.int32),
        pltpu.VMEM((K,), jnp.int32),
        pltpu.VMEM((K,), jnp.int32),
        pltpu.VMEM((K, H), jnp.float32),
        pltpu.VMEM((K, H), jnp.float32),
        pltpu.VMEM_SHARED((NP, H), jnp.float32),
        pltpu.SemaphoreType.DMA,
        pltpu.SemaphoreType.DMA,
        pltpu.SemaphoreType.DMA,
        pltpu.SemaphoreType.DMA,
    ],
)


R = 1024
GRID = NP // R


def _mm_scale_kernel(x_ref, w_ref, d0_ref, d1_ref, hs_ref, dinv_ref):
    deg = d0_ref[...] + d1_ref[...] + 1.0
    dinv = lax.rsqrt(deg)
    h = jnp.dot(x_ref[...], w_ref[...], preferred_element_type=jnp.float32)
    hs_ref[...] = h * dinv
    dinv_ref[...] = jnp.broadcast_to(dinv, (R, H))


def _mm_scale(xp, W1, d0, d1):
    return pl.pallas_call(
        _mm_scale_kernel,
        grid=(GRID,),
        in_specs=[
            pl.BlockSpec((R, D), lambda i: (i, 0)),
            pl.BlockSpec((D, H), lambda i: (0, 0)),
            pl.BlockSpec((R, 1), lambda i: (i, 0)),
            pl.BlockSpec((R, 1), lambda i: (i, 0)),
        ],
        out_specs=[
            pl.BlockSpec((R, H), lambda i: (i, 0)),
            pl.BlockSpec((R, H), lambda i: (i, 0)),
        ],
        out_shape=[
            jax.ShapeDtypeStruct((NP, H), jnp.float32),
            jax.ShapeDtypeStruct((NP, H), jnp.float32),
        ],
    )(xp, W1, d0, d1)


def _mid_kernel(p0_ref, p1_ref, hs_ref, dinv_ref, b1_ref, g2_ref):
    agg = p0_ref[...] + p1_ref[...] + hs_ref[...]
    pre = agg * dinv_ref[...] + b1_ref[...]
    g2_ref[...] = jnp.maximum(pre, 0.0) * dinv_ref[...]


def _mid(p0, p1, hs, dinvB, b1):
    return pl.pallas_call(
        _mid_kernel,
        grid=(GRID,),
        in_specs=[
            pl.BlockSpec((R, H), lambda i: (i, 0)),
            pl.BlockSpec((R, H), lambda i: (i, 0)),
            pl.BlockSpec((R, H), lambda i: (i, 0)),
            pl.BlockSpec((R, H), lambda i: (i, 0)),
            pl.BlockSpec((1, H), lambda i: (0, 0)),
        ],
        out_specs=pl.BlockSpec((R, H), lambda i: (i, 0)),
        out_shape=jax.ShapeDtypeStruct((NP, H), jnp.float32),
    )(p0, p1, hs, dinvB, b1)


def _final_kernel(q0_ref, q1_ref, g2_ref, dinv_ref, w2_ref, b2_ref, out_ref):
    t = (q0_ref[...] + q1_ref[...] + g2_ref[...]) * dinv_ref[...]
    logits = jnp.dot(t, w2_ref[...], preferred_element_type=jnp.float32)
    logits = logits + b2_ref[...]
    m = jnp.max(logits, axis=1, keepdims=True)
    y = logits - m
    lse = jnp.log(jnp.sum(jnp.exp(y), axis=1, keepdims=True))
    out_ref[...] = y - lse


def _final(q0, q1, g2, dinvB, W2, b2):
    return pl.pallas_call(
        _final_kernel,
        grid=(GRID,),
        in_specs=[
            pl.BlockSpec((R, H), lambda i: (i, 0)),
            pl.BlockSpec((R, H), lambda i: (i, 0)),
            pl.BlockSpec((R, H), lambda i: (i, 0)),
            pl.BlockSpec((R, H), lambda i: (i, 0)),
            pl.BlockSpec((H, C), lambda i: (0, 0)),
            pl.BlockSpec((1, C), lambda i: (0, 0)),
        ],
        out_specs=pl.BlockSpec((R, C), lambda i: (i, 0)),
        out_shape=jax.ShapeDtypeStruct((NP, C), jnp.float32),
    )(q0, q1, g2, dinvB, W2, b2)


@jax.jit
def kernel(x, edge_index, W1, b1, W2, b2):
    ei = edge_index.astype(jnp.int32)
    dst2 = ei[1].reshape(NW, NCH, K)
    xp = jnp.pad(x, ((0, NP - N), (0, 0)))

    zerosD = jnp.zeros((NP, H), jnp.float32)

    onesK = jnp.ones((K, H), jnp.float32)
    degp = _deg(dst2, onesK, zerosD)
    d0 = degp[:NP, 0:1]
    d1 = degp[NP:, 0:1]

    hs, dinvB = _mm_scale(xp, W1, d0, d1)

    p = _agg(hs, ei[0], ei[1], zerosD)
    g2 = _mid(p[:NP], p[NP:], hs, dinvB, b1.reshape(1, H))

    q = _agg(g2, ei[0], ei[1], zerosD)
    out = _final(q[:NP], q[NP:], g2, dinvB, W2, b2.reshape(1, C))
    return out[:N]
